# Initial kernel scaffold; baseline (speedup 1.0000x reference)
import jax
import jax.numpy as jnp
from jax.experimental import pallas as pl


def kernel(inputs, embed):
    raise NotImplementedError("write your pallas kernel here")


# speedup vs baseline: 1.1806x; 1.1806x over previous
import functools

import jax
import jax.numpy as jnp
from jax import lax
from jax.experimental import pallas as pl
from jax.experimental.pallas import tpu as pltpu
from jax.experimental.pallas import tpu_sc as plsc

_DIM = 256
_NE = 8192
_TM = 256
_CK = 1024

_NC = 2
_NS = 16
_NW = _NC * _NS
_CH = 128


def _argmin_tile(x_ref, e_ref, e2_ref, idx_ref, dsum_ref):
    x = x_ref[...]
    x2 = jnp.sum(x * x, axis=1, keepdims=True)
    best_d = None
    best_i = None
    for j in range(_NE // _CK):
        e = e_ref[:, j * _CK:(j + 1) * _CK]
        e2 = e2_ref[:, j * _CK:(j + 1) * _CK]
        mm = jnp.dot(x, e, preferred_element_type=jnp.float32)
        d = (x2 - 2.0 * mm) + e2
        m = jnp.min(d, axis=1, keepdims=True)
        ii = lax.broadcasted_iota(jnp.int32, d.shape, 1)
        cand = jnp.min(jnp.where(d == m, ii, _NE), axis=1, keepdims=True) + j * _CK
        if best_d is None:
            best_d, best_i = m, cand
        else:
            better = m < best_d
            best_i = jnp.where(better, cand, best_i)
            best_d = jnp.where(better, m, best_d)
    idx_ref[...] = best_i
    dsum_ref[...] = jnp.sum(best_d, axis=0, keepdims=True).reshape(1, 1, 1)


def _tc_argmin(flat, embed, e2):
    nt = flat.shape[0] // _TM
    idx, dsum = pl.pallas_call(
        _argmin_tile,
        grid=(nt,),
        in_specs=[
            pl.BlockSpec((_TM, _DIM), lambda i: (i, 0)),
            pl.BlockSpec((_DIM, _NE), lambda i: (0, 0)),
            pl.BlockSpec((1, _NE), lambda i: (0, 0)),
        ],
        out_specs=[
            pl.BlockSpec((_TM, 1), lambda i: (i, 0)),
            pl.BlockSpec((1, 1, 1), lambda i: (i, 0, 0)),
        ],
        out_shape=[
            jax.ShapeDtypeStruct((flat.shape[0], 1), jnp.int32),
            jax.ShapeDtypeStruct((nt, 1, 1), jnp.float32),
        ],
    )(flat, embed, e2)
    return idx[:, 0], dsum


def _sc_gather(table, idx):
    B = idx.shape[0]
    bw = B // _NW
    mesh = plsc.VectorSubcoreMesh(core_axis_name="c", subcore_axis_name="s")

    @functools.partial(
        pl.kernel,
        mesh=mesh,
        out_type=jax.ShapeDtypeStruct((B, _DIM), jnp.float32),
        scratch_types=[
            pltpu.VMEM((_CH,), jnp.int32),
            pltpu.VMEM((_CH, _DIM), jnp.float32),
            pltpu.SemaphoreType.DMA,
        ],
    )
    def k(table_hbm, idx_hbm, out_hbm, idx_v, rows_v, sem):
        wid = lax.axis_index("s") * _NC + lax.axis_index("c")
        base = wid * bw
        for c in range(bw // _CH):
            off = base + c * _CH
            pltpu.sync_copy(idx_hbm.at[pl.ds(off, _CH)], idx_v)
            pltpu.async_copy(table_hbm.at[idx_v], rows_v, sem).wait()
            pltpu.sync_copy(rows_v, out_hbm.at[pl.ds(off, _CH)])

    return k(table, idx)


def kernel(inputs, embed):
    flat = inputs.reshape(-1, _DIM)
    e2 = jnp.sum(embed ** 2, axis=0, keepdims=True)
    idx, dsum = _tc_argmin(flat, embed, e2)
    q = _sc_gather(embed.T, idx)
    quantize = q.reshape(inputs.shape)
    diff = jnp.sum(dsum) / (flat.shape[0] * _DIM)
    embed_ind = idx.reshape(inputs.shape[:-1])
    return (quantize, diff, embed_ind)


# speedup vs baseline: 1.5068x; 1.2762x over previous
import functools

import jax
import jax.numpy as jnp
from jax import lax
from jax.experimental import pallas as pl
from jax.experimental.pallas import tpu as pltpu
from jax.experimental.pallas import tpu_sc as plsc

_DIM = 256
_NE = 8192
_TM = 256
_CK = 1024

_NC = 2
_NS = 16
_NW = _NC * _NS
_CH = 128


def _argmin_tile(x_ref, em2_ref, e2_ref, idx_ref, dsum_ref):
    x = x_ref[...]
    x2 = jnp.sum(x * x, axis=1, keepdims=True)
    best_d = None
    best_i = None
    ii = lax.broadcasted_iota(jnp.int32, (_TM, _CK), 1).astype(jnp.float32)
    for j in range(_NE // _CK):
        em2 = em2_ref[:, j * _CK:(j + 1) * _CK]
        e2 = e2_ref[:, j * _CK:(j + 1) * _CK]
        mm2 = jnp.dot(x, em2, preferred_element_type=jnp.float32)
        d = (x2 + mm2) + e2
        m = jnp.min(d, axis=1, keepdims=True)
        cand = jnp.min(jnp.where(d == m, ii, float(_NE)), axis=1, keepdims=True) + float(j * _CK)
        if best_d is None:
            best_d, best_i = m, cand
        else:
            better = m < best_d
            best_i = jnp.where(better, cand, best_i)
            best_d = jnp.where(better, m, best_d)
    idx_ref[...] = best_i.astype(jnp.int32)
    dsum_ref[...] = jnp.sum(best_d, axis=0, keepdims=True).reshape(1, 1, 1)


def _tc_argmin(flat, em2, e2):
    nt = flat.shape[0] // _TM
    idx, dsum = pl.pallas_call(
        _argmin_tile,
        grid=(nt,),
        in_specs=[
            pl.BlockSpec((_TM, _DIM), lambda i: (i, 0)),
            pl.BlockSpec((_DIM, _NE), lambda i: (0, 0)),
            pl.BlockSpec((1, _NE), lambda i: (0, 0)),
        ],
        out_specs=[
            pl.BlockSpec((_TM, 1), lambda i: (i, 0)),
            pl.BlockSpec((1, 1, 1), lambda i: (i, 0, 0)),
        ],
        out_shape=[
            jax.ShapeDtypeStruct((flat.shape[0], 1), jnp.int32),
            jax.ShapeDtypeStruct((nt, 1, 1), jnp.float32),
        ],
    )(flat, em2, e2)
    return idx[:, 0], dsum


def _sc_gather(table, idx):
    B = idx.shape[0]
    bw = B // _NW
    mesh = plsc.VectorSubcoreMesh(core_axis_name="c", subcore_axis_name="s")

    @functools.partial(
        pl.kernel,
        mesh=mesh,
        out_type=jax.ShapeDtypeStruct((B, _DIM), jnp.float32),
        scratch_types=[
            pltpu.VMEM((_CH,), jnp.int32),
            pltpu.VMEM((_CH, _DIM), jnp.float32),
            pltpu.SemaphoreType.DMA,
        ],
    )
    def k(table_hbm, idx_hbm, out_hbm, idx_v, rows_v, sem):
        wid = lax.axis_index("s") * _NC + lax.axis_index("c")
        base = wid * bw
        for c in range(bw // _CH):
            off = base + c * _CH
            pltpu.sync_copy(idx_hbm.at[pl.ds(off, _CH)], idx_v)
            pltpu.async_copy(table_hbm.at[idx_v], rows_v, sem).wait()
            pltpu.sync_copy(rows_v, out_hbm.at[pl.ds(off, _CH)])

    return k(table, idx)


def kernel(inputs, embed):
    flat = inputs.reshape(-1, _DIM)
    e2 = jnp.sum(embed ** 2, axis=0, keepdims=True)
    idx, dsum = _tc_argmin(flat, -2.0 * embed, e2)
    q = _sc_gather(embed.T, idx)
    quantize = q.reshape(inputs.shape)
    diff = jnp.sum(dsum) / (flat.shape[0] * _DIM)
    embed_ind = idx.reshape(inputs.shape[:-1])
    return (quantize, diff, embed_ind)
